# baseline (speedup 1.0000x reference)
import functools

import jax
import jax.numpy as jnp
from jax import lax
from jax.experimental import pallas as pl
from jax.experimental.pallas import tpu as pltpu
from jax.experimental.pallas import tpu_sc as plsc

_DELTA = 2.5
_BN_EPS = 1e-5

_NC = 2
_NS = 16
_NW = _NC * _NS
_L = 16

_BIG = 3.0e38

_C0 = 4096
_FB = 8192
_LCAP = 6 * _FB

_G = 512
_B2 = 128


def _sc_build_lists(src, dst, npad, cap):
    R = npad // _NW
    epad2 = src.shape[0]
    nchunks = epad2 // _C0
    mesh = plsc.VectorSubcoreMesh(core_axis_name="c", subcore_axis_name="s")

    def body(src_hbm, dst_hbm, lists_hbm, counts_hbm, deg_hbm,
             sbufA, dbufA, sbufB, dbufB, lbuf, sbuf2, deg_acc, cur, cbuf,
             semSA, semDA, semSB, semDB):
        wid = lax.axis_index("s") * _NC + lax.axis_index("c")
        lo = wid * R
        izero16 = jnp.zeros((_L,), jnp.int32)
        iones16 = jnp.ones((_L,), jnp.int32)
        sent16 = izero16 + (R << 14)

        def zdeg(i, carry):
            deg_acc[pl.ds(i * _L, _L)] = izero16
            return carry
        lax.fori_loop(0, R // _L, zdeg, 0)

        def start(ci, sbuf, dbuf, semS, semD):
            base = ci * _C0
            pltpu.make_async_copy(src_hbm.at[pl.ds(base, _C0)], sbuf, semS).start()
            pltpu.make_async_copy(dst_hbm.at[pl.ds(base, _C0)], dbuf, semD).start()

        def wait(sbuf, dbuf, semS, semD):
            pltpu.make_async_copy(src_hbm.at[pl.ds(0, _C0)], sbuf, semS).wait()
            pltpu.make_async_copy(dst_hbm.at[pl.ds(0, _C0)], dbuf, semD).wait()

        def filt_chunk(sbuf, dbuf, cnt):
            def filt(i, nm):
                dv = dbuf[pl.ds(i * _L, _L)]
                sv = sbuf[pl.ds(i * _L, _L)]
                m = (dv >= lo) & (dv < lo + R)
                ld = dv - lo
                packed = sv | lax.shift_left(ld, 14)
                pos = nm + plsc.cumsum(jnp.where(m, 1, 0)) - 1
                plsc.store_scatter(lbuf, [pos], packed, mask=m)
                plsc.addupdate_scatter(deg_acc, [ld], iones16, mask=m)
                return jnp.max(pos) + 1
            return lax.fori_loop(0, _C0 // _L, filt, cnt)

        def maybe_flush(cnt, woff):
            do = (cnt >= _FB) & ((woff > 0) | (cnt > _LCAP - _C0))
            @pl.when(do)
            def _():
                woff8 = pl.multiple_of(woff, _FB)
                pltpu.sync_copy(lbuf.at[pl.ds(0, _FB)],
                                lists_hbm.at[pl.ds(wid * cap + woff8, _FB)])
                rem = cnt - _FB
                def mv(i, c2):
                    lbuf[pl.ds(i * _L, _L)] = lbuf[pl.ds(_FB + i * _L, _L)]
                    return c2
                lax.fori_loop(0, (rem + _L - 1) // _L, mv, 0)
            cnt = jnp.where(do, cnt - _FB, cnt)
            woff = jnp.where(do, woff + _FB, woff)
            return cnt, woff

        start(0, sbufA, dbufA, semSA, semDA)
        start(1, sbufB, dbufB, semSB, semDB)

        def body2(k, carry):
            cnt, woff = carry
            wait(sbufA, dbufA, semSA, semDA)
            cnt = filt_chunk(sbufA, dbufA, cnt)
            start(jnp.minimum(2 * k + 2, nchunks - 2), sbufA, dbufA, semSA, semDA)
            cnt, woff = maybe_flush(cnt, woff)
            wait(sbufB, dbufB, semSB, semDB)
            cnt = filt_chunk(sbufB, dbufB, cnt)
            start(jnp.minimum(2 * k + 3, nchunks - 1), sbufB, dbufB, semSB, semDB)
            cnt, woff = maybe_flush(cnt, woff)
            return cnt, woff
        cnt, woff = lax.fori_loop(0, nchunks // 2, body2, (0, 0))
        wait(sbufA, dbufA, semSA, semDA)
        wait(sbufB, dbufB, semSB, semDB)

        lane0 = jnp.arange(_L, dtype=jnp.int32) == 0
        inc0 = jnp.where(lane0, 1, 0)

        @pl.when(woff == 0)
        def _():
            cnt16 = cnt - lax.rem(cnt, _L)
            lanes = jnp.arange(_L, dtype=jnp.int32)
            tv = lbuf[pl.ds(cnt16, _L)]
            lbuf[pl.ds(cnt16, _L)] = jnp.where(lanes < cnt - cnt16, tv, sent16)
            def pfx(i, carry2):
                dv = deg_acc[pl.ds(i * _L, _L)]
                inc = plsc.cumsum(dv) + carry2
                cur[pl.ds(i * _L, _L)] = inc - dv
                return jnp.max(inc)
            lax.fori_loop(0, R // _L, pfx, 0)
            cur[pl.ds(R, _L)] = izero16 + cnt

            def sct(g, c2):
                vv = lbuf[pl.ds(g * _L, _L)]
                for k in range(_L):
                    v = vv[k]
                    ldk = lax.shift_right_logical(v, 14)
                    cvec = cur[pl.ds(ldk, _L)]
                    slot = cvec[0]
                    cur[pl.ds(ldk, _L)] = cvec + inc0
                    plsc.store_scatter(sbuf2, [izero16 + slot], izero16 + v,
                                       mask=lane0)
                return c2
            lax.fori_loop(0, (cnt + _L - 1) // _L, sct, 0)

            def ffl(i, c2):
                pltpu.sync_copy(sbuf2.at[pl.ds(i * _FB, _FB)],
                                lists_hbm.at[pl.ds(wid * cap + i * _FB, _FB)])
                return c2
            lax.fori_loop(0, (cnt + _FB - 1) // _FB, ffl, 0)

        @pl.when(woff > 0)
        def _():
            woff8 = pl.multiple_of(woff, _FB)
            def ufl(i, c2):
                pltpu.sync_copy(lbuf.at[pl.ds(i * _FB, _FB)],
                                lists_hbm.at[pl.ds(wid * cap + woff8 + i * _FB,
                                                   _FB)])
                return c2
            lax.fori_loop(0, (cnt + _FB - 1) // _FB, ufl, 0)

        total = woff + cnt
        sflag = jnp.where(woff == 0, 1, 0)
        lidx = jnp.arange(_L, dtype=jnp.int32)
        cbuf[pl.ds(0, _L)] = jnp.where(
            lidx == 0, total, jnp.where(lidx == 1, sflag, 0))
        pltpu.sync_copy(cbuf, counts_hbm.at[pl.ds(wid * _L, _L)])
        pltpu.sync_copy(deg_acc, deg_hbm.at[pl.ds(lo, R)])

    f = pl.kernel(
        body,
        out_type=(
            jax.ShapeDtypeStruct((_NW * cap,), jnp.int32),
            jax.ShapeDtypeStruct((_NW * _L,), jnp.int32),
            jax.ShapeDtypeStruct((npad,), jnp.int32),
        ),
        mesh=mesh,
        compiler_params=pltpu.CompilerParams(needs_layout_passes=False),
        scratch_types=[
            pltpu.VMEM((_C0,), jnp.int32),
            pltpu.VMEM((_C0,), jnp.int32),
            pltpu.VMEM((_C0,), jnp.int32),
            pltpu.VMEM((_C0,), jnp.int32),
            pltpu.VMEM((_LCAP + _L,), jnp.int32),
            pltpu.VMEM((_LCAP + _L,), jnp.int32),
            pltpu.VMEM((R,), jnp.int32),
            pltpu.VMEM((R + _L,), jnp.int32),
            pltpu.VMEM((_L,), jnp.int32),
            pltpu.SemaphoreType.DMA,
            pltpu.SemaphoreType.DMA,
            pltpu.SemaphoreType.DMA,
            pltpu.SemaphoreType.DMA,
        ],
    )
    return f(src, dst)


def _sc_agg_lists(A, lists, counts, deg, npad, d, cap):
    R = npad // _NW
    R2 = R // 2
    nj = d // _L
    G2 = 256
    mesh = plsc.VectorSubcoreMesh(core_axis_name="c", subcore_axis_name="s")

    def body(a_hbm, lists_hbm, counts_hbm, deg_hbm,
             s_hbm, m_hbm, x_hbm,
             sum_acc, min_acc, max_acc, lbuf, srcs, lds, rows,
             degv, offs, semA, semB):
        wid = lax.axis_index("s") * _NC + lax.axis_index("c")
        lo = wid * R
        zero16 = jnp.zeros((_L,), jnp.float32)

        pltpu.sync_copy(counts_hbm.at[pl.ds(wid * _L, _L)], lds.at[pl.ds(0, _L)])
        cntv = lds[pl.ds(0, _L)]
        total = cntv[0]
        sortedf = cntv[1]
        nch = (total + G2 - 1) // G2

        pltpu.sync_copy(deg_hbm.at[pl.ds(lo, R)], degv.at[pl.ds(0, R)])

        def pfx(i, carry2):
            dv = degv[pl.ds(i * _L, _L)]
            inc = plsc.cumsum(dv) + carry2
            offs[pl.ds(i * _L, _L)] = inc - dv
            return jnp.max(inc)
        lax.fori_loop(0, R // _L, pfx, 0)
        offs[pl.ds(R, _L)] = jnp.zeros((_L,), jnp.int32) + total

        def stage(c, half, sem):
            pltpu.sync_copy(lists_hbm.at[pl.ds(wid * cap + c * G2, G2)],
                            lbuf.at[pl.ds(half, G2)])
            def up(i, cc):
                sl = pl.ds(half + i * _L, _L)
                srcs[sl] = lbuf[sl] & 16383
                return cc
            lax.fori_loop(0, G2 // _L, up, 0)
            pltpu.make_async_copy(
                a_hbm.at[srcs.at[pl.ds(half, 128)]],
                rows.at[pl.ds(half, 128)], sem).start()
            pltpu.make_async_copy(
                a_hbm.at[srcs.at[pl.ds(half + 128, 128)]],
                rows.at[pl.ds(half + 128, 128)], sem).start()

        def waitg(half, sem):
            pltpu.make_async_copy(
                a_hbm.at[srcs.at[pl.ds(0, 128)]],
                rows.at[pl.ds(half, 128)], sem).wait()
            pltpu.make_async_copy(
                a_hbm.at[srcs.at[pl.ds(0, 128)]],
                rows.at[pl.ds(half + 128, 128)], sem).wait()

        def stage_dyn(c):
            @pl.when(lax.rem(c, 2) == 0)
            def _():
                stage(c, 0, semA)
            @pl.when(lax.rem(c, 2) == 1)
            def _():
                stage(c, G2, semB)

        def wait_dyn(c):
            @pl.when(lax.rem(c, 2) == 0)
            def _():
                waitg(0, semA)
            @pl.when(lax.rem(c, 2) == 1)
            def _():
                waitg(G2, semB)

        for q in range(2):
            nb = q * R2

            def zrow(r, carry):
                for j in range(nj):
                    sl = pl.ds(j * _L, _L)
                    sum_acc[r, sl] = zero16
                    min_acc[r, sl] = zero16 + _BIG
                    max_acc[r, sl] = zero16 - _BIG
                return carry
            lax.fori_loop(0, R2, zrow, 0)

            @pl.when(sortedf == 1)
            def _():
                qb = offs[pl.ds(nb, _L)][0]
                qe = offs[pl.ds(nb + R2, _L)][0]
                clast = lax.div(qe - 1, G2)

                @pl.when(qb < qe)
                def _():
                    stage_dyn(lax.div(qb, G2))

                def node(r, carry):
                    gr = nb + r
                    eb = offs[pl.ds(gr, _L)][0]
                    dg = degv[pl.ds(gr, _L)][0]
                    ee = eb + dg

                    def edge(e, regs):
                        @pl.when((e == qb) | (lax.rem(e, G2) == 0))
                        def _():
                            c = lax.div(e, G2)
                            wait_dyn(c)
                            @pl.when(c + 1 <= clast)
                            def _():
                                stage_dyn(c + 1)
                        emod = lax.rem(e, 2 * G2)
                        out = []
                        for j in range(nj):
                            rv = rows[emod, pl.ds(j * _L, _L)]
                            out.append(regs[j] + rv)
                        for j in range(nj):
                            rv = rows[emod, pl.ds(j * _L, _L)]
                            out.append(jnp.minimum(regs[nj + j], rv))
                        for j in range(nj):
                            rv = rows[emod, pl.ds(j * _L, _L)]
                            out.append(jnp.maximum(regs[2 * nj + j], rv))
                        return tuple(out)

                    init = ([zero16] * nj + [zero16 + _BIG] * nj
                            + [zero16 - _BIG] * nj)
                    regs = lax.fori_loop(eb, ee, edge, tuple(init))
                    for j in range(nj):
                        sl = pl.ds(j * _L, _L)
                        sum_acc[r, sl] = regs[j]
                        min_acc[r, sl] = regs[nj + j]
                        max_acc[r, sl] = regs[2 * nj + j]
                    return carry
                lax.fori_loop(0, R2, node, 0)

            @pl.when(sortedf == 0)
            def _():
                def fchunk(ci, carry):
                    goff = ci * G2
                    valid = jnp.minimum(total - goff, G2)
                    pltpu.sync_copy(
                        lists_hbm.at[pl.ds(wid * cap + goff, G2)],
                        lbuf.at[pl.ds(0, G2)])
                    def up2(i, cc):
                        sl = pl.ds(i * _L, _L)
                        v = lbuf[sl]
                        srcs[sl] = v & 16383
                        lds[sl] = lax.shift_right_logical(v, 14)
                        return cc
                    lax.fori_loop(0, G2 // _L, up2, 0)
                    pltpu.async_copy(
                        a_hbm.at[srcs.at[pl.ds(0, 128)]],
                        rows.at[pl.ds(0, 128)], semA).wait()
                    pltpu.async_copy(
                        a_hbm.at[srcs.at[pl.ds(128, 128)]],
                        rows.at[pl.ds(128, 128)], semA).wait()
                    def fe(e, cc):
                        ldr = lds[pl.ds(e, _L)][0]
                        lr = ldr - nb
                        @pl.when((lr >= 0) & (lr < R2))
                        def _():
                            for j in range(nj):
                                sl = pl.ds(j * _L, _L)
                                rv = rows[e, sl]
                                sum_acc[lr, sl] = sum_acc[lr, sl] + rv
                                min_acc[lr, sl] = jnp.minimum(min_acc[lr, sl], rv)
                                max_acc[lr, sl] = jnp.maximum(max_acc[lr, sl], rv)
                        return cc
                    lax.fori_loop(0, valid, fe, 0)
                    return carry
                lax.fori_loop(0, nch, fchunk, 0)

            pltpu.sync_copy(sum_acc, s_hbm.at[pl.ds(lo + nb, R2)])
            pltpu.sync_copy(min_acc, m_hbm.at[pl.ds(lo + nb, R2)])
            pltpu.sync_copy(max_acc, x_hbm.at[pl.ds(lo + nb, R2)])

    f = pl.kernel(
        body,
        out_type=tuple(
            jax.ShapeDtypeStruct((npad, d), jnp.float32) for _ in range(3)),
        mesh=mesh,
        compiler_params=pltpu.CompilerParams(needs_layout_passes=False),
        scratch_types=[
            pltpu.VMEM((R2, d), jnp.float32),
            pltpu.VMEM((R2, d), jnp.float32),
            pltpu.VMEM((R2, d), jnp.float32),
            pltpu.VMEM((2 * 256,), jnp.int32),
            pltpu.VMEM((2 * 256,), jnp.int32),
            pltpu.VMEM((256 + _L,), jnp.int32),
            pltpu.VMEM((2 * 256, d), jnp.float32),
            pltpu.VMEM((R + _L,), jnp.int32),
            pltpu.VMEM((R + _L,), jnp.int32),
            pltpu.SemaphoreType.DMA,
            pltpu.SemaphoreType.DMA,
        ],
    )
    return f(A, lists, counts, deg)


def _tc_pre(h, wa, wb, bias, bm=1024):
    npad, d = h.shape
    dh = d // 2

    def body(h_ref, wa_ref, wb_ref, b_ref, a_ref, bt_ref):
        hb = h_ref[...]
        a_ref[...] = jnp.dot(hb, wa_ref[...], preferred_element_type=jnp.float32)
        bt_ref[...] = (
            jnp.dot(hb, wb_ref[...], preferred_element_type=jnp.float32)
            + b_ref[...])

    return pl.pallas_call(
        body,
        grid=(npad // bm,),
        in_specs=[
            pl.BlockSpec((bm, d), lambda i: (i, 0)),
            pl.BlockSpec((d, d), lambda i: (0, 0)),
            pl.BlockSpec((d, d), lambda i: (0, 0)),
            pl.BlockSpec((1, d), lambda i: (0, 0)),
        ],
        out_specs=[
            pl.BlockSpec((bm, d), lambda i: (i, 0)),
            pl.BlockSpec((bm, d), lambda i: (i, 0)),
        ],
        out_shape=[
            jax.ShapeDtypeStruct((npad, d), jnp.float32),
            jax.ShapeDtypeStruct((npad, d), jnp.float32),
        ],
    )(h, wa, wb, bias)


def _tc_post(h, sums, mins, maxs, deg, bt, w1, b1, residual, relu,
             bm=1024):
    npad, d = h.shape
    do = w1.shape[1]

    def body(h_ref, s_ref, mn_ref, mx_ref, dg_ref, bt_ref, w_ref, b_ref,
             o_ref):
        hb = h_ref[...]
        btb = bt_ref[...]
        dg = dg_ref[...].astype(jnp.float32)
        has = dg > 0.0
        dinv = 1.0 / jnp.maximum(dg, 1.0)
        mean = jnp.where(has, s_ref[...] * dinv + btb, 0.0)
        mn = jnp.where(has, mn_ref[...] + btb, 0.0)
        mx = jnp.where(has, mx_ref[...] + btb, 0.0)
        logd = jnp.log(dg + 1.0)
        amp = jnp.where(has, logd / _DELTA, 0.0)
        att = jnp.where(has, _DELTA / jnp.maximum(logd, 1e-12), 0.0)
        hcat = jnp.concatenate(
            [hb, mean, mn, mx,
             mean * amp, mn * amp, mx * amp,
             mean * att, mn * att, mx * att], axis=1)
        out = jnp.dot(hcat, w_ref[...], preferred_element_type=jnp.float32) + b_ref[...]
        out = jnp.where(out > 0.0, out, 0.01 * out)
        if residual:
            out = out + hb
        if relu:
            out = jnp.maximum(out, 0.0)
        o_ref[...] = out

    full = pl.BlockSpec((bm, d), lambda i: (i, 0))
    return pl.pallas_call(
        body,
        grid=(npad // bm,),
        in_specs=[
            full, full, full, full,
            pl.BlockSpec((bm, 1), lambda i: (i, 0)),
            full,
            pl.BlockSpec((10 * d, do), lambda i: (0, 0)),
            pl.BlockSpec((1, do), lambda i: (0, 0)),
        ],
        out_specs=pl.BlockSpec((bm, do), lambda i: (i, 0)),
        out_shape=jax.ShapeDtypeStruct((npad, do), jnp.float32),
    )(h, sums, mins, maxs, deg, bt, w1, b1)


def _round_up(v, m):
    return (v + m - 1) // m * m


def kernel(x, edge_index, params):
    n, d_in = x.shape
    e = edge_index.shape[1]
    npad = _round_up(n, _NW * _L)
    epad2 = _round_up(e, 2 * _C0)
    cap = _round_up(epad2, _FB) + 2 * _FB

    src = edge_index[0]
    dst = edge_index[1]
    src_p = jnp.concatenate([src, jnp.zeros((epad2 - e,), jnp.int32)])
    dst_p = jnp.concatenate([dst, jnp.full((epad2 - e,), npad, jnp.int32)])

    lists, counts, deg = _sc_build_lists(src_p, dst_p, npad, cap)

    h = jnp.pad(x, ((0, npad - n), (0, 0)))
    nlayers = len(params)
    for li, layer in enumerate(params):
        M_w, M_b, U_w, U_b, bn_g, bn_b, mix_w, mix_b = layer
        di = M_w.shape[0]
        do = U_w.shape[0]
        wa = M_w[:, :di].T
        wb = M_w[:, di:].T
        A, Bt = _tc_pre(h, wa, wb, M_b[None, :])
        sums, mins, maxs = _sc_agg_lists(A, lists, counts, deg,
                                         npad, di, cap)
        s = bn_g / jnp.sqrt(1.0 + _BN_EPS)
        w1 = U_w.T @ (s[:, None] * mix_w.T)
        b1 = (U_b * s + bn_b) @ mix_w.T + mix_b
        h = _tc_post(h, sums, mins, maxs, deg[:, None], Bt,
                     w1, b1[None, :],
                     residual=(di == do), relu=(li != nlayers - 1))
    return h[:n]


# speedup vs baseline: 1.0287x; 1.0287x over previous
import functools

import jax
import jax.numpy as jnp
from jax import lax
from jax.experimental import pallas as pl
from jax.experimental.pallas import tpu as pltpu
from jax.experimental.pallas import tpu_sc as plsc

_DELTA = 2.5
_BN_EPS = 1e-5

_NC = 2
_NS = 16
_NW = _NC * _NS
_L = 16

_BIG = 3.0e38

_C0 = 4096
_FB = 8192
_LCAP = 6 * _FB

_G = 512
_B2 = 128


def _sc_build_lists(src, dst, npad, cap):
    R = npad // _NW
    epad2 = src.shape[0]
    nchunks = epad2 // _C0
    mesh = plsc.VectorSubcoreMesh(core_axis_name="c", subcore_axis_name="s")

    def body(src_hbm, dst_hbm, lists_hbm, counts_hbm, deg_hbm,
             sbufA, dbufA, sbufB, dbufB, lbuf, sbuf2, deg_acc, cur, cbuf,
             semSA, semDA, semSB, semDB):
        wid = lax.axis_index("s") * _NC + lax.axis_index("c")
        lo = wid * R
        izero16 = jnp.zeros((_L,), jnp.int32)
        iones16 = jnp.ones((_L,), jnp.int32)
        sent16 = izero16 + (R << 14)

        def zdeg(i, carry):
            deg_acc[pl.ds(i * _L, _L)] = izero16
            return carry
        lax.fori_loop(0, R // _L, zdeg, 0)

        def start(ci, sbuf, dbuf, semS, semD):
            base = ci * _C0
            pltpu.make_async_copy(src_hbm.at[pl.ds(base, _C0)], sbuf, semS).start()
            pltpu.make_async_copy(dst_hbm.at[pl.ds(base, _C0)], dbuf, semD).start()

        def wait(sbuf, dbuf, semS, semD):
            pltpu.make_async_copy(src_hbm.at[pl.ds(0, _C0)], sbuf, semS).wait()
            pltpu.make_async_copy(dst_hbm.at[pl.ds(0, _C0)], dbuf, semD).wait()

        def filt_chunk(sbuf, dbuf, cnt):
            def filt(i, nm):
                dv = dbuf[pl.ds(i * _L, _L)]
                sv = sbuf[pl.ds(i * _L, _L)]
                m = (dv >= lo) & (dv < lo + R)
                ld = dv - lo
                packed = sv | lax.shift_left(ld, 14)
                pos = nm + plsc.cumsum(jnp.where(m, 1, 0)) - 1
                plsc.store_scatter(lbuf, [pos], packed, mask=m)
                plsc.addupdate_scatter(deg_acc, [ld], iones16, mask=m)
                cnt16 = plsc.all_reduce_population_count(m)
                return nm + jnp.max(cnt16)
            return lax.fori_loop(0, _C0 // _L, filt, cnt)

        def maybe_flush(cnt, woff):
            do = (cnt >= _FB) & ((woff > 0) | (cnt > _LCAP - _C0))
            @pl.when(do)
            def _():
                woff8 = pl.multiple_of(woff, _FB)
                pltpu.sync_copy(lbuf.at[pl.ds(0, _FB)],
                                lists_hbm.at[pl.ds(wid * cap + woff8, _FB)])
                rem = cnt - _FB
                def mv(i, c2):
                    lbuf[pl.ds(i * _L, _L)] = lbuf[pl.ds(_FB + i * _L, _L)]
                    return c2
                lax.fori_loop(0, (rem + _L - 1) // _L, mv, 0)
            cnt = jnp.where(do, cnt - _FB, cnt)
            woff = jnp.where(do, woff + _FB, woff)
            return cnt, woff

        start(0, sbufA, dbufA, semSA, semDA)
        start(1, sbufB, dbufB, semSB, semDB)

        def body2(k, carry):
            cnt, woff = carry
            wait(sbufA, dbufA, semSA, semDA)
            cnt = filt_chunk(sbufA, dbufA, cnt)
            start(jnp.minimum(2 * k + 2, nchunks - 2), sbufA, dbufA, semSA, semDA)
            cnt, woff = maybe_flush(cnt, woff)
            wait(sbufB, dbufB, semSB, semDB)
            cnt = filt_chunk(sbufB, dbufB, cnt)
            start(jnp.minimum(2 * k + 3, nchunks - 1), sbufB, dbufB, semSB, semDB)
            cnt, woff = maybe_flush(cnt, woff)
            return cnt, woff
        cnt, woff = lax.fori_loop(0, nchunks // 2, body2, (0, 0))
        wait(sbufA, dbufA, semSA, semDA)
        wait(sbufB, dbufB, semSB, semDB)

        lane0 = jnp.arange(_L, dtype=jnp.int32) == 0
        inc0 = jnp.where(lane0, 1, 0)

        @pl.when(woff == 0)
        def _():
            cnt16 = cnt - lax.rem(cnt, _L)
            lanes = jnp.arange(_L, dtype=jnp.int32)
            tv = lbuf[pl.ds(cnt16, _L)]
            lbuf[pl.ds(cnt16, _L)] = jnp.where(lanes < cnt - cnt16, tv, sent16)
            def pfx(i, carry2):
                dv = deg_acc[pl.ds(i * _L, _L)]
                inc = plsc.cumsum(dv) + carry2
                cur[pl.ds(i * _L, _L)] = inc - dv
                return jnp.max(inc)
            lax.fori_loop(0, R // _L, pfx, 0)
            cur[pl.ds(R, _L)] = izero16 + cnt

            def sct(g, c2):
                vv = lbuf[pl.ds(g * _L, _L)]
                for k in range(_L):
                    v = vv[k]
                    ldk = lax.shift_right_logical(v, 14)
                    cvec = cur[pl.ds(ldk, _L)]
                    slot = cvec[0]
                    cur[pl.ds(ldk, _L)] = cvec + inc0
                    plsc.store_scatter(sbuf2, [izero16 + slot], izero16 + v,
                                       mask=lane0)
                return c2
            lax.fori_loop(0, (cnt + _L - 1) // _L, sct, 0)

            def ffl(i, c2):
                pltpu.sync_copy(sbuf2.at[pl.ds(i * _FB, _FB)],
                                lists_hbm.at[pl.ds(wid * cap + i * _FB, _FB)])
                return c2
            lax.fori_loop(0, (cnt + _FB - 1) // _FB, ffl, 0)

        @pl.when(woff > 0)
        def _():
            woff8 = pl.multiple_of(woff, _FB)
            def ufl(i, c2):
                pltpu.sync_copy(lbuf.at[pl.ds(i * _FB, _FB)],
                                lists_hbm.at[pl.ds(wid * cap + woff8 + i * _FB,
                                                   _FB)])
                return c2
            lax.fori_loop(0, (cnt + _FB - 1) // _FB, ufl, 0)

        total = woff + cnt
        sflag = jnp.where(woff == 0, 1, 0)
        lidx = jnp.arange(_L, dtype=jnp.int32)
        cbuf[pl.ds(0, _L)] = jnp.where(
            lidx == 0, total, jnp.where(lidx == 1, sflag, 0))
        pltpu.sync_copy(cbuf, counts_hbm.at[pl.ds(wid * _L, _L)])
        pltpu.sync_copy(deg_acc, deg_hbm.at[pl.ds(lo, R)])

    f = pl.kernel(
        body,
        out_type=(
            jax.ShapeDtypeStruct((_NW * cap,), jnp.int32),
            jax.ShapeDtypeStruct((_NW * _L,), jnp.int32),
            jax.ShapeDtypeStruct((npad,), jnp.int32),
        ),
        mesh=mesh,
        compiler_params=pltpu.CompilerParams(needs_layout_passes=False),
        scratch_types=[
            pltpu.VMEM((_C0,), jnp.int32),
            pltpu.VMEM((_C0,), jnp.int32),
            pltpu.VMEM((_C0,), jnp.int32),
            pltpu.VMEM((_C0,), jnp.int32),
            pltpu.VMEM((_LCAP + _L,), jnp.int32),
            pltpu.VMEM((_LCAP + _L,), jnp.int32),
            pltpu.VMEM((R,), jnp.int32),
            pltpu.VMEM((R + _L,), jnp.int32),
            pltpu.VMEM((_L,), jnp.int32),
            pltpu.SemaphoreType.DMA,
            pltpu.SemaphoreType.DMA,
            pltpu.SemaphoreType.DMA,
            pltpu.SemaphoreType.DMA,
        ],
    )
    return f(src, dst)


def _sc_agg_lists(A, lists, counts, deg, npad, d, cap):
    R = npad // _NW
    R2 = R // 2
    nj = d // _L
    G2 = 256
    mesh = plsc.VectorSubcoreMesh(core_axis_name="c", subcore_axis_name="s")

    def body(a_hbm, lists_hbm, counts_hbm, deg_hbm,
             s_hbm, m_hbm, x_hbm,
             sum_acc, min_acc, max_acc, lbuf, srcs, lds, rows,
             degv, offs, semA, semB):
        wid = lax.axis_index("s") * _NC + lax.axis_index("c")
        lo = wid * R
        zero16 = jnp.zeros((_L,), jnp.float32)

        pltpu.sync_copy(counts_hbm.at[pl.ds(wid * _L, _L)], lds.at[pl.ds(0, _L)])
        cntv = lds[pl.ds(0, _L)]
        total = cntv[0]
        sortedf = cntv[1]
        nch = (total + G2 - 1) // G2

        pltpu.sync_copy(deg_hbm.at[pl.ds(lo, R)], degv.at[pl.ds(0, R)])

        def pfx(i, carry2):
            dv = degv[pl.ds(i * _L, _L)]
            inc = plsc.cumsum(dv) + carry2
            offs[pl.ds(i * _L, _L)] = inc - dv
            return jnp.max(inc)
        lax.fori_loop(0, R // _L, pfx, 0)
        offs[pl.ds(R, _L)] = jnp.zeros((_L,), jnp.int32) + total

        def stage(c, half, sem):
            pltpu.sync_copy(lists_hbm.at[pl.ds(wid * cap + c * G2, G2)],
                            lbuf.at[pl.ds(half, G2)])
            def up(i, cc):
                sl = pl.ds(half + i * _L, _L)
                srcs[sl] = lbuf[sl] & 16383
                return cc
            lax.fori_loop(0, G2 // _L, up, 0)
            pltpu.make_async_copy(
                a_hbm.at[srcs.at[pl.ds(half, 128)]],
                rows.at[pl.ds(half, 128)], sem).start()
            pltpu.make_async_copy(
                a_hbm.at[srcs.at[pl.ds(half + 128, 128)]],
                rows.at[pl.ds(half + 128, 128)], sem).start()

        def waitg(half, sem):
            pltpu.make_async_copy(
                a_hbm.at[srcs.at[pl.ds(0, 128)]],
                rows.at[pl.ds(half, 128)], sem).wait()
            pltpu.make_async_copy(
                a_hbm.at[srcs.at[pl.ds(0, 128)]],
                rows.at[pl.ds(half + 128, 128)], sem).wait()

        def stage_dyn(c):
            @pl.when(lax.rem(c, 2) == 0)
            def _():
                stage(c, 0, semA)
            @pl.when(lax.rem(c, 2) == 1)
            def _():
                stage(c, G2, semB)

        def wait_dyn(c):
            @pl.when(lax.rem(c, 2) == 0)
            def _():
                waitg(0, semA)
            @pl.when(lax.rem(c, 2) == 1)
            def _():
                waitg(G2, semB)

        for q in range(2):
            nb = q * R2

            def zrow(r, carry):
                for j in range(nj):
                    sl = pl.ds(j * _L, _L)
                    sum_acc[r, sl] = zero16
                    min_acc[r, sl] = zero16 + _BIG
                    max_acc[r, sl] = zero16 - _BIG
                return carry
            lax.fori_loop(0, R2, zrow, 0)

            @pl.when(sortedf == 1)
            def _():
                qb = offs[pl.ds(nb, _L)][0]
                qe = offs[pl.ds(nb + R2, _L)][0]
                clast = lax.div(qe - 1, G2)

                @pl.when(qb < qe)
                def _():
                    stage_dyn(lax.div(qb, G2))

                def node(r, carry):
                    gr = nb + r
                    eb = offs[pl.ds(gr, _L)][0]
                    dg = degv[pl.ds(gr, _L)][0]
                    ee = eb + dg

                    def edge(e, regs):
                        @pl.when((e == qb) | (lax.rem(e, G2) == 0))
                        def _():
                            c = lax.div(e, G2)
                            wait_dyn(c)
                            @pl.when(c + 1 <= clast)
                            def _():
                                stage_dyn(c + 1)
                        emod = lax.rem(e, 2 * G2)
                        out = []
                        for j in range(nj):
                            rv = rows[emod, pl.ds(j * _L, _L)]
                            out.append(regs[j] + rv)
                        for j in range(nj):
                            rv = rows[emod, pl.ds(j * _L, _L)]
                            out.append(jnp.minimum(regs[nj + j], rv))
                        for j in range(nj):
                            rv = rows[emod, pl.ds(j * _L, _L)]
                            out.append(jnp.maximum(regs[2 * nj + j], rv))
                        return tuple(out)

                    init = ([zero16] * nj + [zero16 + _BIG] * nj
                            + [zero16 - _BIG] * nj)
                    regs = lax.fori_loop(eb, ee, edge, tuple(init))
                    for j in range(nj):
                        sl = pl.ds(j * _L, _L)
                        sum_acc[r, sl] = regs[j]
                        min_acc[r, sl] = regs[nj + j]
                        max_acc[r, sl] = regs[2 * nj + j]
                    return carry
                lax.fori_loop(0, R2, node, 0)

            @pl.when(sortedf == 0)
            def _():
                def fchunk(ci, carry):
                    goff = ci * G2
                    valid = jnp.minimum(total - goff, G2)
                    pltpu.sync_copy(
                        lists_hbm.at[pl.ds(wid * cap + goff, G2)],
                        lbuf.at[pl.ds(0, G2)])
                    def up2(i, cc):
                        sl = pl.ds(i * _L, _L)
                        v = lbuf[sl]
                        srcs[sl] = v & 16383
                        lds[sl] = lax.shift_right_logical(v, 14)
                        return cc
                    lax.fori_loop(0, G2 // _L, up2, 0)
                    pltpu.async_copy(
                        a_hbm.at[srcs.at[pl.ds(0, 128)]],
                        rows.at[pl.ds(0, 128)], semA).wait()
                    pltpu.async_copy(
                        a_hbm.at[srcs.at[pl.ds(128, 128)]],
                        rows.at[pl.ds(128, 128)], semA).wait()
                    def fe(e, cc):
                        ldr = lds[pl.ds(e, _L)][0]
                        lr = ldr - nb
                        @pl.when((lr >= 0) & (lr < R2))
                        def _():
                            for j in range(nj):
                                sl = pl.ds(j * _L, _L)
                                rv = rows[e, sl]
                                sum_acc[lr, sl] = sum_acc[lr, sl] + rv
                                min_acc[lr, sl] = jnp.minimum(min_acc[lr, sl], rv)
                                max_acc[lr, sl] = jnp.maximum(max_acc[lr, sl], rv)
                        return cc
                    lax.fori_loop(0, valid, fe, 0)
                    return carry
                lax.fori_loop(0, nch, fchunk, 0)

            pltpu.sync_copy(sum_acc, s_hbm.at[pl.ds(lo + nb, R2)])
            pltpu.sync_copy(min_acc, m_hbm.at[pl.ds(lo + nb, R2)])
            pltpu.sync_copy(max_acc, x_hbm.at[pl.ds(lo + nb, R2)])

    f = pl.kernel(
        body,
        out_type=tuple(
            jax.ShapeDtypeStruct((npad, d), jnp.float32) for _ in range(3)),
        mesh=mesh,
        compiler_params=pltpu.CompilerParams(needs_layout_passes=False),
        scratch_types=[
            pltpu.VMEM((R2, d), jnp.float32),
            pltpu.VMEM((R2, d), jnp.float32),
            pltpu.VMEM((R2, d), jnp.float32),
            pltpu.VMEM((2 * 256,), jnp.int32),
            pltpu.VMEM((2 * 256,), jnp.int32),
            pltpu.VMEM((256 + _L,), jnp.int32),
            pltpu.VMEM((2 * 256, d), jnp.float32),
            pltpu.VMEM((R + _L,), jnp.int32),
            pltpu.VMEM((R + _L,), jnp.int32),
            pltpu.SemaphoreType.DMA,
            pltpu.SemaphoreType.DMA,
        ],
    )
    return f(A, lists, counts, deg)


def _tc_pre(h, wa, wb, bias, bm=1024):
    npad, d = h.shape
    dh = d // 2

    def body(h_ref, wa_ref, wb_ref, b_ref, a_ref, bt_ref):
        hb = h_ref[...]
        a_ref[...] = jnp.dot(hb, wa_ref[...], preferred_element_type=jnp.float32)
        bt_ref[...] = (
            jnp.dot(hb, wb_ref[...], preferred_element_type=jnp.float32)
            + b_ref[...])

    return pl.pallas_call(
        body,
        grid=(npad // bm,),
        in_specs=[
            pl.BlockSpec((bm, d), lambda i: (i, 0)),
            pl.BlockSpec((d, d), lambda i: (0, 0)),
            pl.BlockSpec((d, d), lambda i: (0, 0)),
            pl.BlockSpec((1, d), lambda i: (0, 0)),
        ],
        out_specs=[
            pl.BlockSpec((bm, d), lambda i: (i, 0)),
            pl.BlockSpec((bm, d), lambda i: (i, 0)),
        ],
        out_shape=[
            jax.ShapeDtypeStruct((npad, d), jnp.float32),
            jax.ShapeDtypeStruct((npad, d), jnp.float32),
        ],
    )(h, wa, wb, bias)


def _tc_post(h, sums, mins, maxs, deg, bt, w1, b1, residual, relu,
             bm=1024):
    npad, d = h.shape
    do = w1.shape[1]

    def body(h_ref, s_ref, mn_ref, mx_ref, dg_ref, bt_ref, w_ref, b_ref,
             o_ref):
        hb = h_ref[...]
        btb = bt_ref[...]
        dg = dg_ref[...].astype(jnp.float32)
        has = dg > 0.0
        dinv = 1.0 / jnp.maximum(dg, 1.0)
        mean = jnp.where(has, s_ref[...] * dinv + btb, 0.0)
        mn = jnp.where(has, mn_ref[...] + btb, 0.0)
        mx = jnp.where(has, mx_ref[...] + btb, 0.0)
        logd = jnp.log(dg + 1.0)
        amp = jnp.where(has, logd / _DELTA, 0.0)
        att = jnp.where(has, _DELTA / jnp.maximum(logd, 1e-12), 0.0)
        hcat = jnp.concatenate(
            [hb, mean, mn, mx,
             mean * amp, mn * amp, mx * amp,
             mean * att, mn * att, mx * att], axis=1)
        out = jnp.dot(hcat, w_ref[...], preferred_element_type=jnp.float32) + b_ref[...]
        out = jnp.where(out > 0.0, out, 0.01 * out)
        if residual:
            out = out + hb
        if relu:
            out = jnp.maximum(out, 0.0)
        o_ref[...] = out

    full = pl.BlockSpec((bm, d), lambda i: (i, 0))
    return pl.pallas_call(
        body,
        grid=(npad // bm,),
        in_specs=[
            full, full, full, full,
            pl.BlockSpec((bm, 1), lambda i: (i, 0)),
            full,
            pl.BlockSpec((10 * d, do), lambda i: (0, 0)),
            pl.BlockSpec((1, do), lambda i: (0, 0)),
        ],
        out_specs=pl.BlockSpec((bm, do), lambda i: (i, 0)),
        out_shape=jax.ShapeDtypeStruct((npad, do), jnp.float32),
    )(h, sums, mins, maxs, deg, bt, w1, b1)


def _round_up(v, m):
    return (v + m - 1) // m * m


def kernel(x, edge_index, params):
    n, d_in = x.shape
    e = edge_index.shape[1]
    npad = _round_up(n, _NW * _L)
    epad2 = _round_up(e, 2 * _C0)
    cap = _round_up(epad2, _FB) + 2 * _FB

    src = edge_index[0]
    dst = edge_index[1]
    src_p = jnp.concatenate([src, jnp.zeros((epad2 - e,), jnp.int32)])
    dst_p = jnp.concatenate([dst, jnp.full((epad2 - e,), npad, jnp.int32)])

    lists, counts, deg = _sc_build_lists(src_p, dst_p, npad, cap)

    h = jnp.pad(x, ((0, npad - n), (0, 0)))
    nlayers = len(params)
    for li, layer in enumerate(params):
        M_w, M_b, U_w, U_b, bn_g, bn_b, mix_w, mix_b = layer
        di = M_w.shape[0]
        do = U_w.shape[0]
        wa = M_w[:, :di].T
        wb = M_w[:, di:].T
        A, Bt = _tc_pre(h, wa, wb, M_b[None, :])
        sums, mins, maxs = _sc_agg_lists(A, lists, counts, deg,
                                         npad, di, cap)
        s = bn_g / jnp.sqrt(1.0 + _BN_EPS)
        w1 = U_w.T @ (s[:, None] * mix_w.T)
        b1 = (U_b * s + bn_b) @ mix_w.T + mix_b
        h = _tc_post(h, sums, mins, maxs, deg[:, None], Bt,
                     w1, b1[None, :],
                     residual=(di == do), relu=(li != nlayers - 1))
    return h[:n]
